# baseline (speedup 1.0000x reference)
import functools

import jax
import jax.numpy as jnp
from jax import lax
from jax.experimental import pallas as pl
from jax.experimental.pallas import tpu as pltpu
from jax.experimental.pallas import tpu_sc as plsc

B = 1024
AD = 512
NF = 64
NLB = 128
NACT = 256
HID = 1024
IN1 = NLB + AD * NF + AD

NW = 32
SPW = B // NW

R = B * AD
RT = 4096
NRT = R // RT

NKP = 256
NK = 66


def _sc_counts(active_as):
    mesh = plsc.VectorSubcoreMesh(core_axis_name="c", subcore_axis_name="s")

    @functools.partial(
        pl.kernel,
        out_type=(jax.ShapeDtypeStruct((B * AD,), jnp.float32),
                  jax.ShapeDtypeStruct((NKP, B, 2), jnp.float32)),
        mesh=mesh,
        compiler_params=pltpu.CompilerParams(needs_layout_passes=False,
                                             use_tc_tiling_on_sc=False),
        scratch_types=[
            pltpu.VMEM((SPW * NACT,), jnp.int32),
            pltpu.VMEM((SPW * AD,), jnp.float32),
            pltpu.VMEM((NKP, SPW, 2), jnp.float32),
        ],
    )
    def k(act_hbm, cnt_hbm, mr_hbm, act_v, cnt_v, m_v):
        wid = lax.axis_index("s") * 2 + lax.axis_index("c")
        base = wid * SPW
        pltpu.sync_copy(act_hbm.at[pl.ds(wid * SPW * NACT, SPW * NACT)],
                        act_v)
        zeros16 = jnp.zeros((16,), jnp.float32)
        ones16 = jnp.ones((16,), jnp.float32)

        def zero_body(i, _):
            cnt_v[pl.ds(i * 16, 16)] = zeros16
            return 0

        lax.fori_loop(0, SPW * AD // 16, zero_body, 0)

        def scat_body(s, _):
            srow = jnp.full((16,), s * AD, jnp.int32)
            for v in range(NACT // 16):
                idx = act_v[pl.ds(s * NACT + v * 16, 16)]
                plsc.addupdate_scatter(cnt_v, [srow + idx], ones16)
            return 0

        lax.fori_loop(0, SPW, scat_body, 0)

        tt = lax.broadcasted_iota(jnp.int32, (16,), 0)

        def mask_body(s, _):
            svec = jnp.full((16,), s, jnp.int32)
            for v in range(AD // 16):
                cnt = cnt_v[pl.ds(s * AD + v * 16, 16)]
                msk = jnp.minimum(cnt, 1.0)
                a = tt + (v * 16)
                plsc.store_scatter(
                    m_v, [lax.shift_right_logical(a, 1), svec,
                          lax.bitwise_and(a, 1)], msk)
            return 0

        lax.fori_loop(0, SPW, mask_body, 0)
        pltpu.sync_copy(cnt_v, cnt_hbm.at[pl.ds(wid * SPW * AD, SPW * AD)])
        pltpu.sync_copy(m_v, mr_hbm.at[:, pl.ds(base, SPW), :])

    return k(active_as.reshape(B * NACT))


def _stats_body(c_ref, f_ref, flb_ref, st_ref, acc):
    r = pl.program_id(0)

    @pl.when(r == 0)
    def _():
        acc[...] = jnp.zeros((8, NF), jnp.float32)

    cb = c_ref[0]
    fb = f_ref[...]
    dn = (((1,), (0,)), ((), ()))
    s1 = lax.dot_general(cb, fb, dn, precision=lax.Precision.HIGHEST,
                         preferred_element_type=jnp.float32)
    s2 = lax.dot_general(cb, fb * fb, dn, precision=lax.Precision.HIGHEST,
                         preferred_element_type=jnp.float32)
    acc[0:1, :] += s1
    acc[1:2, :] += s2

    @pl.when(r == NRT - 1)
    def _():
        flb = flb_ref[...]
        head = flb[:, :NF]
        tail = flb[:, NF:]
        acc[2:3, :] += jnp.sum(head, axis=0, keepdims=True)
        acc[3:4, :] += jnp.sum(head * head, axis=0, keepdims=True)
        acc[4:5, :] += jnp.sum(tail, axis=0, keepdims=True)
        acc[5:6, :] += jnp.sum(tail * tail, axis=0, keepdims=True)
        st_ref[...] = acc[...]


def _stats_call(c3, f_r, flb):
    return pl.pallas_call(
        _stats_body,
        grid=(NRT,),
        in_specs=[
            pl.BlockSpec((1, 1, RT), lambda r: (r, 0, 0)),
            pl.BlockSpec((RT, NF), lambda r: (r, 0)),
            pl.BlockSpec((B, NLB), lambda r: (0, 0)),
        ],
        out_specs=pl.BlockSpec((8, NF), lambda r: (0, 0)),
        out_shape=jax.ShapeDtypeStruct((8, NF), jnp.float32),
        scratch_shapes=[pltpu.VMEM((8, NF), jnp.float32)],
    )(c3, f_r, flb)


def _mm_body(f0, f1, f2, f3, m0, m1, m2, m3, flb, act, alb, blb, w1, at, bt,
             b1r, lnw, lnb, w4, b4r, out_ref, acc):
    k = pl.program_id(0)

    @pl.when(k == 0)
    def _():
        acc[...] = jnp.zeros((B, HID), jnp.float32)

    def piece(f_ref, m_ref):
        m2d = m_ref[0]
        mex = jnp.concatenate(
            [jnp.broadcast_to(m2d[:, 0:1], (B, NF)),
             jnp.broadcast_to(m2d[:, 1:2], (B, NF))], axis=1)
        return mex * (f_ref[...] * at[...] + bt[...])

    dn = (((1,), (0,)), ((), ()))

    def accum(x, w):
        acc[...] += lax.dot_general(x.astype(jnp.bfloat16),
                                    w.astype(jnp.bfloat16), dn,
                                    preferred_element_type=jnp.float32)

    @pl.when(k == 0)
    def _():
        x0 = jnp.concatenate(
            [flb[:, NF:] * alb[...] + blb[...],
             flb[:, :NF] * at[:, :NF] + bt[:, :NF]], axis=1)
        x = jnp.concatenate([x0, piece(f1, m1), piece(f2, m2),
                             piece(f3, m3)], axis=1)
        accum(x, w1[...])

    @pl.when((k >= 1) & (k <= 63))
    def _():
        x = jnp.concatenate([piece(f0, m0), piece(f1, m1), piece(f2, m2),
                             piece(f3, m3)], axis=1)
        accum(x, w1[...])

    @pl.when(k == 64)
    def _():
        x = jnp.concatenate([piece(f0, m0), act[:, 0:384]], axis=1)
        accum(x, w1[...])

    @pl.when(k == NK - 1)
    def _():
        accum(act[:, 384:512], w1[0:128, :])
        h = acc[...] + b1r[...]
        h = jnp.where(h > 0, h, jnp.exp(jnp.minimum(h, 0.0)) - 1.0)
        mu = jnp.mean(h, axis=1, keepdims=True)
        hc = h - mu
        var = jnp.mean(hc * hc, axis=1, keepdims=True)
        hn = hc * lax.rsqrt(var + 1e-5) * lnw[...] + lnb[...]
        out_ref[...] = lax.dot_general(
            hn.astype(jnp.bfloat16), w4[...].astype(jnp.bfloat16), dn,
            preferred_element_type=jnp.float32) + b4r[...]


def _mm_call(f2d, m_r, flb, act, alb, blb, W1, at, bt, b1r, lnw, lnb, W4,
             b4r):
    def fmap(j):
        return lambda k: (0, jnp.clip(4 * k - 1 + j, 0, NKP - 1))

    def mmap(j):
        return lambda k: (jnp.clip(4 * k - 1 + j, 0, NKP - 1), 0, 0)

    def full(shape):
        return pl.BlockSpec(shape, lambda k: tuple(0 for _ in shape))

    return pl.pallas_call(
        _mm_body,
        grid=(NK,),
        in_specs=[
            pl.BlockSpec((B, 128), fmap(0)),
            pl.BlockSpec((B, 128), fmap(1)),
            pl.BlockSpec((B, 128), fmap(2)),
            pl.BlockSpec((B, 128), fmap(3)),
            pl.BlockSpec((1, B, 2), mmap(0)),
            pl.BlockSpec((1, B, 2), mmap(1)),
            pl.BlockSpec((1, B, 2), mmap(2)),
            pl.BlockSpec((1, B, 2), mmap(3)),
            full((B, NLB)),
            full((B, AD)),
            full((1, NF)),
            full((1, NF)),
            pl.BlockSpec((512, HID), lambda k: (k, 0)),
            full((1, 128)),
            full((1, 128)),
            full((1, HID)),
            full((1, HID)),
            full((1, HID)),
            full((HID, AD)),
            full((1, AD)),
        ],
        out_specs=pl.BlockSpec((B, AD), lambda k: (0, 0)),
        out_shape=jax.ShapeDtypeStruct((B, AD), jnp.float32),
        scratch_shapes=[pltpu.VMEM((B, HID), jnp.float32)],
    )(f2d, f2d, f2d, f2d, m_r, m_r, m_r, m_r, flb, act, alb, blb, W1, at, bt,
      b1r, lnw, lnb, W4, b4r)


def kernel(feature_lb, feature_as, action, active_as, bn_as_w, bn_as_b,
           bn_lb_w, bn_lb_b, W1, b1, ln1_w, ln1_b, W4, b4):
    c, m_r = _sc_counts(active_as)

    f_r = feature_as.reshape(R, NF)
    c3 = c.reshape(NRT, 1, RT)
    st = _stats_call(c3, f_r, feature_lb)

    n_as = jnp.float32(B + B * NACT)
    mean_as = (st[0] + st[2]) / n_as
    var_as = (st[1] + st[3]) / n_as - mean_as * mean_as
    alpha_as = bn_as_w * lax.rsqrt(var_as + 1e-5)
    beta_as = bn_as_b - mean_as * alpha_as

    mean_lb = st[4] / B
    var_lb = st[5] / B - mean_lb * mean_lb
    alpha_lb = bn_lb_w * lax.rsqrt(var_lb + 1e-5)
    beta_lb = bn_lb_b - mean_lb * alpha_lb

    at = jnp.tile(alpha_as, 2)[None, :]
    bt = jnp.tile(beta_as, 2)[None, :]

    f2d = feature_as.reshape(B, AD * NF)
    return _mm_call(f2d, m_r, feature_lb, action, alpha_lb[None, :],
                    beta_lb[None, :], W1, at, bt, b1[None, :],
                    ln1_w[None, :], ln1_b[None, :], W4, b4[None, :])


# speedup vs baseline: 1.2775x; 1.2775x over previous
import functools

import jax
import jax.numpy as jnp
from jax import lax
from jax.experimental import pallas as pl
from jax.experimental.pallas import tpu as pltpu
from jax.experimental.pallas import tpu_sc as plsc

B = 1024
AD = 512
NF = 64
NLB = 128
NACT = 256
HID = 1024
IN1 = NLB + AD * NF + AD

NW = 32
SPW = B // NW

R = B * AD
RT = 4096
NRT = R // RT

NKP = 256
NK = 66


def _sc_counts(active_as):
    mesh = plsc.VectorSubcoreMesh(core_axis_name="c", subcore_axis_name="s")

    @functools.partial(
        pl.kernel,
        out_type=(jax.ShapeDtypeStruct((AD, B), jnp.float32),
                  jax.ShapeDtypeStruct((NKP, B, 2), jnp.float32)),
        mesh=mesh,
        compiler_params=pltpu.CompilerParams(needs_layout_passes=False,
                                             use_tc_tiling_on_sc=False),
        scratch_types=[
            pltpu.VMEM((SPW, NACT), jnp.int32),
            pltpu.VMEM((AD, SPW), jnp.float32),
            pltpu.VMEM((NKP, SPW, 2), jnp.float32),
        ],
    )
    def k(act_hbm, ct_hbm, mr_hbm, act_v, cnt_v, m_v):
        wid = lax.axis_index("s") * 2 + lax.axis_index("c")
        base = wid * SPW
        pltpu.sync_copy(act_hbm.at[pl.ds(base, SPW)], act_v)
        zeros16 = jnp.zeros((16,), jnp.float32)
        ones16 = jnp.ones((16,), jnp.float32)

        def zero_body(a, _):
            cnt_v[a, pl.ds(0, 16)] = zeros16
            cnt_v[a, pl.ds(16, 16)] = zeros16
            return 0

        lax.fori_loop(0, AD, zero_body, 0)

        def scat_body(s, _):
            svec = jnp.full((16,), s, jnp.int32)
            for v in range(NACT // 16):
                idx = act_v[s, pl.ds(v * 16, 16)]
                plsc.addupdate_scatter(cnt_v, [idx, svec], ones16)
            return 0

        lax.fori_loop(0, SPW, scat_body, 0)

        tt = lax.broadcasted_iota(jnp.int32, (16,), 0)

        def mask_body(a, _):
            pvec = jnp.full((16,), lax.shift_right_logical(a, 1), jnp.int32)
            cvec = jnp.full((16,), lax.bitwise_and(a, 1), jnp.int32)
            for h in range(SPW // 16):
                cnt = cnt_v[a, pl.ds(h * 16, 16)]
                msk = jnp.minimum(cnt, 1.0)
                plsc.store_scatter(m_v, [pvec, tt + (h * 16), cvec], msk)
            return 0

        lax.fori_loop(0, AD, mask_body, 0)
        pltpu.sync_copy(cnt_v, ct_hbm.at[:, pl.ds(base, SPW)])
        pltpu.sync_copy(m_v, mr_hbm.at[:, pl.ds(base, SPW), :])

    return k(active_as)


NST = AD // 8


def _stats_body(ct_ref, f_ref, flb_ref, st_ref, a1, a2):
    r = pl.program_id(0)

    @pl.when(r == 0)
    def _():
        a1[...] = jnp.zeros((8, AD), jnp.float32)
        a2[...] = jnp.zeros((8, AD), jnp.float32)

    fb = f_ref[...]
    ct = ct_ref[...]
    dn = (((1,), (0,)), ((), ()))
    a1[...] += lax.dot_general(ct, fb, dn, precision=lax.Precision.HIGHEST,
                               preferred_element_type=jnp.float32)
    a2[...] += lax.dot_general(ct, fb * fb, dn,
                               precision=lax.Precision.HIGHEST,
                               preferred_element_type=jnp.float32)

    @pl.when(r == NST - 1)
    def _():
        s1 = jnp.zeros((1, NF), jnp.float32)
        s2 = jnp.zeros((1, NF), jnp.float32)
        for j in range(8):
            s1 = s1 + a1[j:j + 1, j * NF:(j + 1) * NF]
            s2 = s2 + a2[j:j + 1, j * NF:(j + 1) * NF]
        flb = flb_ref[...]
        head = flb[:, :NF]
        tail = flb[:, NF:]
        st_ref[0:1, :] = s1
        st_ref[1:2, :] = s2
        st_ref[2:3, :] = jnp.sum(head, axis=0, keepdims=True)
        st_ref[3:4, :] = jnp.sum(head * head, axis=0, keepdims=True)
        st_ref[4:5, :] = jnp.sum(tail, axis=0, keepdims=True)
        st_ref[5:6, :] = jnp.sum(tail * tail, axis=0, keepdims=True)
        st_ref[6:8, :] = jnp.zeros((2, NF), jnp.float32)


def _stats_call(ct, f2d, flb):
    return pl.pallas_call(
        _stats_body,
        grid=(NST,),
        in_specs=[
            pl.BlockSpec((8, B), lambda r: (r, 0)),
            pl.BlockSpec((B, AD), lambda r: (0, r)),
            pl.BlockSpec((B, NLB), lambda r: (0, 0)),
        ],
        out_specs=pl.BlockSpec((8, NF), lambda r: (0, 0)),
        out_shape=jax.ShapeDtypeStruct((8, NF), jnp.float32),
        scratch_shapes=[pltpu.VMEM((8, AD), jnp.float32),
                        pltpu.VMEM((8, AD), jnp.float32)],
    )(ct, f2d, flb)


def _mm_body(f0, f1, f2, f3, m0, m1, m2, m3, flb, act, alb, blb, w1, at, bt,
             b1r, lnw, lnb, w4, b4r, out_ref, acc):
    k = pl.program_id(0)

    @pl.when(k == 0)
    def _():
        acc[...] = jnp.zeros((B, HID), jnp.float32)

    def piece(f_ref, m_ref):
        m2d = m_ref[0]
        mex = jnp.concatenate(
            [jnp.broadcast_to(m2d[:, 0:1], (B, NF)),
             jnp.broadcast_to(m2d[:, 1:2], (B, NF))], axis=1)
        return mex * (f_ref[...] * at[...] + bt[...])

    dn = (((1,), (0,)), ((), ()))

    def accum(x, w):
        acc[...] += lax.dot_general(x.astype(jnp.bfloat16),
                                    w.astype(jnp.bfloat16), dn,
                                    preferred_element_type=jnp.float32)

    @pl.when(k == 0)
    def _():
        x0 = jnp.concatenate(
            [flb[:, NF:] * alb[...] + blb[...],
             flb[:, :NF] * at[:, :NF] + bt[:, :NF]], axis=1)
        x = jnp.concatenate([x0, piece(f1, m1), piece(f2, m2),
                             piece(f3, m3)], axis=1)
        accum(x, w1[...])

    @pl.when((k >= 1) & (k <= 63))
    def _():
        x = jnp.concatenate([piece(f0, m0), piece(f1, m1), piece(f2, m2),
                             piece(f3, m3)], axis=1)
        accum(x, w1[...])

    @pl.when(k == 64)
    def _():
        x = jnp.concatenate([piece(f0, m0), act[:, 0:384]], axis=1)
        accum(x, w1[...])

    @pl.when(k == NK - 1)
    def _():
        accum(act[:, 384:512], w1[0:128, :])
        h = acc[...] + b1r[...]
        h = jnp.where(h > 0, h, jnp.exp(jnp.minimum(h, 0.0)) - 1.0)
        mu = jnp.mean(h, axis=1, keepdims=True)
        hc = h - mu
        var = jnp.mean(hc * hc, axis=1, keepdims=True)
        hn = hc * lax.rsqrt(var + 1e-5) * lnw[...] + lnb[...]
        out_ref[...] = lax.dot_general(
            hn.astype(jnp.bfloat16), w4[...].astype(jnp.bfloat16), dn,
            preferred_element_type=jnp.float32) + b4r[...]


def _mm_call(f2d, m_r, flb, act, alb, blb, W1, at, bt, b1r, lnw, lnb, W4,
             b4r):
    def fmap(j):
        return lambda k: (0, jnp.clip(4 * k - 1 + j, 0, NKP - 1))

    def mmap(j):
        return lambda k: (jnp.clip(4 * k - 1 + j, 0, NKP - 1), 0, 0)

    def full(shape):
        return pl.BlockSpec(shape, lambda k: tuple(0 for _ in shape))

    return pl.pallas_call(
        _mm_body,
        grid=(NK,),
        in_specs=[
            pl.BlockSpec((B, 128), fmap(0)),
            pl.BlockSpec((B, 128), fmap(1)),
            pl.BlockSpec((B, 128), fmap(2)),
            pl.BlockSpec((B, 128), fmap(3)),
            pl.BlockSpec((1, B, 2), mmap(0)),
            pl.BlockSpec((1, B, 2), mmap(1)),
            pl.BlockSpec((1, B, 2), mmap(2)),
            pl.BlockSpec((1, B, 2), mmap(3)),
            full((B, NLB)),
            full((B, AD)),
            full((1, NF)),
            full((1, NF)),
            pl.BlockSpec((512, HID), lambda k: (k, 0)),
            full((1, 128)),
            full((1, 128)),
            full((1, HID)),
            full((1, HID)),
            full((1, HID)),
            full((HID, AD)),
            full((1, AD)),
        ],
        out_specs=pl.BlockSpec((B, AD), lambda k: (0, 0)),
        out_shape=jax.ShapeDtypeStruct((B, AD), jnp.float32),
        scratch_shapes=[pltpu.VMEM((B, HID), jnp.float32)],
    )(f2d, f2d, f2d, f2d, m_r, m_r, m_r, m_r, flb, act, alb, blb, W1, at, bt,
      b1r, lnw, lnb, W4, b4r)


def kernel(feature_lb, feature_as, action, active_as, bn_as_w, bn_as_b,
           bn_lb_w, bn_lb_b, W1, b1, ln1_w, ln1_b, W4, b4):
    ct, m_r = _sc_counts(active_as)

    f2d = feature_as.reshape(B, AD * NF)
    st = _stats_call(ct, f2d, feature_lb)

    n_as = jnp.float32(B + B * NACT)
    mean_as = (st[0] + st[2]) / n_as
    var_as = (st[1] + st[3]) / n_as - mean_as * mean_as
    alpha_as = bn_as_w * lax.rsqrt(var_as + 1e-5)
    beta_as = bn_as_b - mean_as * alpha_as

    mean_lb = st[4] / B
    var_lb = st[5] / B - mean_lb * mean_lb
    alpha_lb = bn_lb_w * lax.rsqrt(var_lb + 1e-5)
    beta_lb = bn_lb_b - mean_lb * alpha_lb

    at = jnp.tile(alpha_as, 2)[None, :]
    bt = jnp.tile(beta_as, 2)[None, :]

    return _mm_call(f2d, m_r, feature_lb, action, alpha_lb[None, :],
                    beta_lb[None, :], W1, at, bt, b1[None, :],
                    ln1_w[None, :], ln1_b[None, :], W4, b4[None, :])


# speedup vs baseline: 1.8194x; 1.4243x over previous
import functools

import jax
import jax.numpy as jnp
from jax import lax
from jax.experimental import pallas as pl
from jax.experimental.pallas import tpu as pltpu
from jax.experimental.pallas import tpu_sc as plsc

B = 1024
AD = 512
NF = 64
NLB = 128
NACT = 256
HID = 1024
IN1 = NLB + AD * NF + AD
RW1 = IN1 // NF

NW = 32
SPW = B // NW

NK = 66


def _sc_counts(active_as):
    mesh = plsc.VectorSubcoreMesh(core_axis_name="c", subcore_axis_name="s")

    @functools.partial(
        pl.kernel,
        out_type=jax.ShapeDtypeStruct((B, AD), jnp.float32),
        mesh=mesh,
        compiler_params=pltpu.CompilerParams(needs_layout_passes=False,
                                             use_tc_tiling_on_sc=False),
        scratch_types=[
            pltpu.VMEM((SPW, NACT), jnp.int32),
            pltpu.VMEM((SPW, AD), jnp.float32),
        ],
    )
    def k(act_hbm, cnt_hbm, act_v, cnt_v):
        wid = lax.axis_index("s") * 2 + lax.axis_index("c")
        base = wid * SPW
        pltpu.sync_copy(act_hbm.at[pl.ds(base, SPW)], act_v)
        zeros16 = jnp.zeros((16,), jnp.float32)
        ones16 = jnp.ones((16,), jnp.float32)

        def zero_body(s, _):
            for v in range(AD // 16):
                cnt_v[s, pl.ds(v * 16, 16)] = zeros16
            return 0

        lax.fori_loop(0, SPW, zero_body, 0)

        def scat_body(s, _):
            svec = jnp.full((16,), s, jnp.int32)
            for v in range(NACT // 16):
                idx = act_v[s, pl.ds(v * 16, 16)]
                plsc.addupdate_scatter(cnt_v, [svec, idx], ones16)
            return 0

        lax.fori_loop(0, SPW, scat_body, 0)
        pltpu.sync_copy(cnt_v, cnt_hbm.at[pl.ds(base, SPW)])

    return k(active_as)


def _stats_body(c_ref, ft_ref, flb_ref, r1_ref, r2_ref, lb_ref):
    q = pl.program_id(0)
    fb = ft_ref[...]
    cb = c_ref[...]
    t = cb * fb
    r1_ref[0] = jnp.sum(t, axis=0, keepdims=True)
    r2_ref[0] = jnp.sum(t * fb, axis=0, keepdims=True)

    @pl.when(q == NF - 1)
    def _():
        flb = flb_ref[...]
        lb_ref[0:1, :] = jnp.sum(flb, axis=0, keepdims=True)
        lb_ref[1:2, :] = jnp.sum(flb * flb, axis=0, keepdims=True)


def _stats_call(c2d, ft2, flb):
    return pl.pallas_call(
        _stats_body,
        grid=(NF,),
        in_specs=[
            pl.BlockSpec((B, AD), lambda q: (0, 0)),
            pl.BlockSpec((B, AD), lambda q: (0, q)),
            pl.BlockSpec((B, NLB), lambda q: (0, 0)),
        ],
        out_specs=[
            pl.BlockSpec((1, 1, AD), lambda q: (q, 0, 0)),
            pl.BlockSpec((1, 1, AD), lambda q: (q, 0, 0)),
            pl.BlockSpec((2, NLB), lambda q: (0, 0)),
        ],
        out_shape=[
            jax.ShapeDtypeStruct((NF, 1, AD), jnp.float32),
            jax.ShapeDtypeStruct((NF, 1, AD), jnp.float32),
            jax.ShapeDtypeStruct((2, NLB), jnp.float32),
        ],
    )(c2d, ft2, flb)


def _mm_body(asm, bsm, ft, m, flb, act, alb, blb, ahd, bhd, b1r, lnw, lnb,
             w4, b4r, w1r, out_ref, acc, wbuf, wlb, wact, sems, semlb,
             semact):
    k = pl.program_id(0)
    dn = (((1,), (0,)), ((), ()))

    def wk_copy(kk, slot):
        return pltpu.make_async_copy(
            w1r.at[pl.ds(2, AD), pl.ds(kk, 1), :], wbuf.at[slot],
            sems.at[slot])

    @pl.when(k == 0)
    def _():
        acc[...] = jnp.zeros((B, HID), jnp.float32)
        wk_copy(0, 0).start()
        wk_copy(1, 1).start()
        pltpu.make_async_copy(w1r.at[pl.ds(0, 2), :, :], wlb, semlb).start()
        pltpu.make_async_copy(w1r.at[pl.ds(RW1 - 8, 8), :, :], wact,
                              semact).start()

    @pl.when(k <= NF - 1)
    def _():
        slot = lax.rem(k, 2)
        wk_copy(k, slot).wait()
        a = asm[k]
        b = bsm[k]
        x = m[...] * (ft[...] * a + b)
        wv = wbuf[slot].reshape(AD, HID)
        acc[...] += lax.dot_general(x.astype(jnp.bfloat16),
                                    wv.astype(jnp.bfloat16), dn,
                                    preferred_element_type=jnp.float32)

        @pl.when(k <= NF - 3)
        def _():
            wk_copy(k + 2, slot).start()

    @pl.when(k == NF)
    def _():
        pltpu.make_async_copy(w1r.at[pl.ds(0, 2), :, :], wlb, semlb).wait()
        x0 = jnp.concatenate(
            [flb[:, NF:] * alb[...] + blb[...],
             flb[:, :NF] * ahd[...] + bhd[...]], axis=1)
        wv = wlb[...].reshape(NLB, HID)
        acc[...] += lax.dot_general(x0.astype(jnp.bfloat16),
                                    wv.astype(jnp.bfloat16), dn,
                                    preferred_element_type=jnp.float32)

    @pl.when(k == NK - 1)
    def _():
        pltpu.make_async_copy(w1r.at[pl.ds(RW1 - 8, 8), :, :], wact,
                              semact).wait()
        wv = wact[...].reshape(AD, HID)
        acc[...] += lax.dot_general(act[...].astype(jnp.bfloat16),
                                    wv.astype(jnp.bfloat16), dn,
                                    preferred_element_type=jnp.float32)
        h = acc[...] + b1r[...]
        h = jnp.where(h > 0, h, jnp.exp(jnp.minimum(h, 0.0)) - 1.0)
        mu = jnp.mean(h, axis=1, keepdims=True)
        hc = h - mu
        var = jnp.mean(hc * hc, axis=1, keepdims=True)
        hn = hc * lax.rsqrt(var + 1e-5) * lnw[...] + lnb[...]
        out_ref[...] = lax.dot_general(
            hn.astype(jnp.bfloat16), w4[...].astype(jnp.bfloat16), dn,
            preferred_element_type=jnp.float32) + b4r[...]


def _mm_call(alpha, beta, ft2, m, flb, act, alb, blb, ahd, bhd, b1r, lnw,
             lnb, W4, b4r, W1r):
    def full(shape):
        return pl.BlockSpec(shape, lambda k: tuple(0 for _ in shape))

    return pl.pallas_call(
        _mm_body,
        grid=(NK,),
        in_specs=[
            pl.BlockSpec(memory_space=pltpu.SMEM),
            pl.BlockSpec(memory_space=pltpu.SMEM),
            pl.BlockSpec((B, AD), lambda k: (0, jnp.minimum(k, NF - 1))),
            full((B, AD)),
            full((B, NLB)),
            full((B, AD)),
            full((1, NF)),
            full((1, NF)),
            full((1, NF)),
            full((1, NF)),
            full((1, HID)),
            full((1, HID)),
            full((1, HID)),
            full((HID, AD)),
            full((1, AD)),
            pl.BlockSpec(memory_space=pl.ANY),
        ],
        out_specs=pl.BlockSpec((B, AD), lambda k: (0, 0)),
        out_shape=jax.ShapeDtypeStruct((B, AD), jnp.float32),
        scratch_shapes=[
            pltpu.VMEM((B, HID), jnp.float32),
            pltpu.VMEM((2, AD, 1, HID), jnp.float32),
            pltpu.VMEM((2, NF, HID), jnp.float32),
            pltpu.VMEM((8, NF, HID), jnp.float32),
            pltpu.SemaphoreType.DMA((2,)),
            pltpu.SemaphoreType.DMA,
            pltpu.SemaphoreType.DMA,
        ],
    )(alpha, beta, ft2, m, flb, act, alb, blb, ahd, bhd, b1r, lnw, lnb,
      W4, b4r, W1r)


def kernel(feature_lb, feature_as, action, active_as, bn_as_w, bn_as_b,
           bn_lb_w, bn_lb_b, W1, b1, ln1_w, ln1_b, W4, b4):
    c2d = _sc_counts(active_as)

    ft2 = jnp.transpose(feature_as, (0, 2, 1)).reshape(B, NF * AD)
    r1, r2, lbs = _stats_call(c2d, ft2, feature_lb)

    S1 = jnp.sum(r1.reshape(NF, AD), axis=1)
    S2 = jnp.sum(r2.reshape(NF, AD), axis=1)
    n_as = jnp.float32(B + B * NACT)
    mean_as = (S1 + lbs[0, :NF]) / n_as
    var_as = (S2 + lbs[1, :NF]) / n_as - mean_as * mean_as
    alpha_as = bn_as_w * lax.rsqrt(var_as + 1e-5)
    beta_as = bn_as_b - mean_as * alpha_as

    mean_lb = lbs[0, NF:] / B
    var_lb = lbs[1, NF:] / B - mean_lb * mean_lb
    alpha_lb = bn_lb_w * lax.rsqrt(var_lb + 1e-5)
    beta_lb = bn_lb_b - mean_lb * alpha_lb

    m = jnp.minimum(c2d, 1.0)
    W1r = W1.reshape(RW1, NF, HID)

    return _mm_call(alpha_as, beta_as, ft2, m, feature_lb, action,
                    alpha_lb[None, :], beta_lb[None, :], alpha_as[None, :],
                    beta_as[None, :], b1[None, :], ln1_w[None, :],
                    ln1_b[None, :], W4, b4[None, :], W1r)


# speedup vs baseline: 2.0882x; 1.1477x over previous
import functools

import jax
import jax.numpy as jnp
from jax import lax
from jax.experimental import pallas as pl
from jax.experimental.pallas import tpu as pltpu
from jax.experimental.pallas import tpu_sc as plsc

B = 1024
AD = 512
NF = 64
NLB = 128
NACT = 256
HID = 1024
IN1 = NLB + AD * NF + AD
RW1 = IN1 // NF

NW = 32
SPW = B // NW

NK = 66


def _sc_counts(active_as):
    mesh = plsc.VectorSubcoreMesh(core_axis_name="c", subcore_axis_name="s")

    @functools.partial(
        pl.kernel,
        out_type=jax.ShapeDtypeStruct((B, AD), jnp.float32),
        mesh=mesh,
        compiler_params=pltpu.CompilerParams(needs_layout_passes=False,
                                             use_tc_tiling_on_sc=False),
        scratch_types=[
            pltpu.VMEM((SPW, NACT), jnp.int32),
            pltpu.VMEM((SPW, AD), jnp.float32),
        ],
    )
    def k(act_hbm, cnt_hbm, act_v, cnt_v):
        wid = lax.axis_index("s") * 2 + lax.axis_index("c")
        base = wid * SPW
        pltpu.sync_copy(act_hbm.at[pl.ds(base, SPW)], act_v)
        zeros16 = jnp.zeros((16,), jnp.float32)
        ones16 = jnp.ones((16,), jnp.float32)

        def zero_body(s, _):
            for v in range(AD // 16):
                cnt_v[s, pl.ds(v * 16, 16)] = zeros16
            return 0

        lax.fori_loop(0, SPW, zero_body, 0)

        def scat_body(s, _):
            svec = jnp.full((16,), s, jnp.int32)
            for v in range(NACT // 16):
                idx = act_v[s, pl.ds(v * 16, 16)]
                plsc.addupdate_scatter(cnt_v, [svec, idx], ones16)
            return 0

        lax.fori_loop(0, SPW, scat_body, 0)
        pltpu.sync_copy(cnt_v, cnt_hbm.at[pl.ds(base, SPW)])

    return k(active_as)


def _stats_body(c_ref, ft3, flb_ref, r1_ref, r2_ref, lb_ref, ftbuf, sems):
    q = pl.program_id(0)

    def ft_copy(qq, slot):
        return pltpu.make_async_copy(ft3.at[:, pl.ds(qq, 1), :],
                                     ftbuf.at[slot], sems.at[slot])

    @pl.when(q == 0)
    def _():
        ft_copy(0, 0).start()
        ft_copy(1, 1).start()

    slot = lax.rem(q, 2)
    ft_copy(q, slot).wait()
    fb = ftbuf[slot].reshape(B, AD)
    cb = c_ref[...]
    t = cb * fb
    r1_ref[0] = jnp.sum(t, axis=0, keepdims=True)
    r2_ref[0] = jnp.sum(t * fb, axis=0, keepdims=True)

    @pl.when(q <= NF - 3)
    def _():
        ft_copy(q + 2, slot).start()

    @pl.when(q == NF - 1)
    def _():
        flb = flb_ref[...]
        lb_ref[0:1, :] = jnp.sum(flb, axis=0, keepdims=True)
        lb_ref[1:2, :] = jnp.sum(flb * flb, axis=0, keepdims=True)


def _stats_call(c2d, ft3, flb):
    return pl.pallas_call(
        _stats_body,
        grid=(NF,),
        in_specs=[
            pl.BlockSpec((B, AD), lambda q: (0, 0)),
            pl.BlockSpec(memory_space=pl.ANY),
            pl.BlockSpec((B, NLB), lambda q: (0, 0)),
        ],
        out_specs=[
            pl.BlockSpec((1, 1, AD), lambda q: (q, 0, 0)),
            pl.BlockSpec((1, 1, AD), lambda q: (q, 0, 0)),
            pl.BlockSpec((2, NLB), lambda q: (0, 0)),
        ],
        out_shape=[
            jax.ShapeDtypeStruct((NF, 1, AD), jnp.float32),
            jax.ShapeDtypeStruct((NF, 1, AD), jnp.float32),
            jax.ShapeDtypeStruct((2, NLB), jnp.float32),
        ],
        scratch_shapes=[
            pltpu.VMEM((2, B, 1, AD), jnp.float32),
            pltpu.SemaphoreType.DMA((2,)),
        ],
    )(c2d, ft3, flb)


def _mm_body(asm, bsm, ft3, m, flb, act, alb, blb, ahd, bhd, b1r, lnw, lnb,
             w4, b4r, w1r, out_ref, acc, wbuf, ftbuf, wlb, wact, sems,
             ftsems, semlb, semact):
    k = pl.program_id(0)
    dn = (((1,), (0,)), ((), ()))

    def wk_copy(kk, slot):
        return pltpu.make_async_copy(
            w1r.at[pl.ds(2, AD), pl.ds(kk, 1), :], wbuf.at[slot],
            sems.at[slot])

    def ft_copy(kk, slot):
        return pltpu.make_async_copy(ft3.at[:, pl.ds(kk, 1), :],
                                     ftbuf.at[slot], ftsems.at[slot])

    @pl.when(k == 0)
    def _():
        acc[...] = jnp.zeros((B, HID), jnp.float32)
        wk_copy(0, 0).start()
        wk_copy(1, 1).start()
        ft_copy(0, 0).start()
        ft_copy(1, 1).start()
        pltpu.make_async_copy(w1r.at[pl.ds(0, 2), :, :], wlb, semlb).start()
        pltpu.make_async_copy(w1r.at[pl.ds(RW1 - 8, 8), :, :], wact,
                              semact).start()

    @pl.when(k <= NF - 1)
    def _():
        slot = lax.rem(k, 2)
        wk_copy(k, slot).wait()
        ft_copy(k, slot).wait()
        a = asm[k]
        b = bsm[k]
        x = m[...] * (ftbuf[slot].reshape(B, AD) * a + b)
        wv = wbuf[slot].reshape(AD, HID)
        acc[...] += lax.dot_general(x.astype(jnp.bfloat16),
                                    wv.astype(jnp.bfloat16), dn,
                                    preferred_element_type=jnp.float32)

        @pl.when(k <= NF - 3)
        def _():
            wk_copy(k + 2, slot).start()
            ft_copy(k + 2, slot).start()

    @pl.when(k == NF)
    def _():
        pltpu.make_async_copy(w1r.at[pl.ds(0, 2), :, :], wlb, semlb).wait()
        x0 = jnp.concatenate(
            [flb[:, NF:] * alb[...] + blb[...],
             flb[:, :NF] * ahd[...] + bhd[...]], axis=1)
        wv = wlb[...].reshape(NLB, HID)
        acc[...] += lax.dot_general(x0.astype(jnp.bfloat16),
                                    wv.astype(jnp.bfloat16), dn,
                                    preferred_element_type=jnp.float32)

    @pl.when(k == NK - 1)
    def _():
        pltpu.make_async_copy(w1r.at[pl.ds(RW1 - 8, 8), :, :], wact,
                              semact).wait()
        wv = wact[...].reshape(AD, HID)
        acc[...] += lax.dot_general(act[...].astype(jnp.bfloat16),
                                    wv.astype(jnp.bfloat16), dn,
                                    preferred_element_type=jnp.float32)
        h = acc[...] + b1r[...]
        h = jnp.where(h > 0, h, jnp.exp(jnp.minimum(h, 0.0)) - 1.0)
        mu = jnp.mean(h, axis=1, keepdims=True)
        hc = h - mu
        var = jnp.mean(hc * hc, axis=1, keepdims=True)
        hn = hc * lax.rsqrt(var + 1e-5) * lnw[...] + lnb[...]
        out_ref[...] = lax.dot_general(
            hn.astype(jnp.bfloat16), w4[...].astype(jnp.bfloat16), dn,
            preferred_element_type=jnp.float32) + b4r[...]


def _mm_call(alpha, beta, ft3, m, flb, act, alb, blb, ahd, bhd, b1r, lnw,
             lnb, W4, b4r, W1r):
    def full(shape):
        return pl.BlockSpec(shape, lambda k: tuple(0 for _ in shape))

    return pl.pallas_call(
        _mm_body,
        grid=(NK,),
        in_specs=[
            pl.BlockSpec(memory_space=pltpu.SMEM),
            pl.BlockSpec(memory_space=pltpu.SMEM),
            pl.BlockSpec(memory_space=pl.ANY),
            full((B, AD)),
            full((B, NLB)),
            full((B, AD)),
            full((1, NF)),
            full((1, NF)),
            full((1, NF)),
            full((1, NF)),
            full((1, HID)),
            full((1, HID)),
            full((1, HID)),
            full((HID, AD)),
            full((1, AD)),
            pl.BlockSpec(memory_space=pl.ANY),
        ],
        out_specs=pl.BlockSpec((B, AD), lambda k: (0, 0)),
        out_shape=jax.ShapeDtypeStruct((B, AD), jnp.float32),
        scratch_shapes=[
            pltpu.VMEM((B, HID), jnp.float32),
            pltpu.VMEM((2, AD, 1, HID), jnp.float32),
            pltpu.VMEM((2, B, 1, AD), jnp.float32),
            pltpu.VMEM((2, NF, HID), jnp.float32),
            pltpu.VMEM((8, NF, HID), jnp.float32),
            pltpu.SemaphoreType.DMA((2,)),
            pltpu.SemaphoreType.DMA((2,)),
            pltpu.SemaphoreType.DMA,
            pltpu.SemaphoreType.DMA,
        ],
    )(alpha, beta, ft3, m, flb, act, alb, blb, ahd, bhd, b1r, lnw, lnb,
      W4, b4r, W1r)


def kernel(feature_lb, feature_as, action, active_as, bn_as_w, bn_as_b,
           bn_lb_w, bn_lb_b, W1, b1, ln1_w, ln1_b, W4, b4):
    c2d = _sc_counts(active_as)

    ft3 = jnp.transpose(feature_as, (0, 2, 1))
    r1, r2, lbs = _stats_call(c2d, ft3, feature_lb)

    S1 = jnp.sum(r1.reshape(NF, AD), axis=1)
    S2 = jnp.sum(r2.reshape(NF, AD), axis=1)
    n_as = jnp.float32(B + B * NACT)
    mean_as = (S1 + lbs[0, :NF]) / n_as
    var_as = (S2 + lbs[1, :NF]) / n_as - mean_as * mean_as
    alpha_as = bn_as_w * lax.rsqrt(var_as + 1e-5)
    beta_as = bn_as_b - mean_as * alpha_as

    mean_lb = lbs[0, NF:] / B
    var_lb = lbs[1, NF:] / B - mean_lb * mean_lb
    alpha_lb = bn_lb_w * lax.rsqrt(var_lb + 1e-5)
    beta_lb = bn_lb_b - mean_lb * alpha_lb

    m = jnp.minimum(c2d, 1.0)
    W1r = W1.reshape(RW1, NF, HID)

    return _mm_call(alpha_as, beta_as, ft3, m, feature_lb, action,
                    alpha_lb[None, :], beta_lb[None, :], alpha_as[None, :],
                    beta_as[None, :], b1[None, :], ln1_w[None, :],
                    ln1_b[None, :], W4, b4[None, :], W1r)


# speedup vs baseline: 2.2640x; 1.0842x over previous
import functools

import jax
import jax.numpy as jnp
from jax import lax
from jax.experimental import pallas as pl
from jax.experimental.pallas import tpu as pltpu
from jax.experimental.pallas import tpu_sc as plsc

B = 1024
AD = 512
NF = 64
NLB = 128
NACT = 256
HID = 1024
IN1 = NLB + AD * NF + AD
RW1 = IN1 // NF

NW = 32
SPW = B // NW

NK = 66


def _sc_counts(active_as):
    mesh = plsc.VectorSubcoreMesh(core_axis_name="c", subcore_axis_name="s")

    @functools.partial(
        pl.kernel,
        out_type=jax.ShapeDtypeStruct((B, AD), jnp.float32),
        mesh=mesh,
        compiler_params=pltpu.CompilerParams(needs_layout_passes=False,
                                             use_tc_tiling_on_sc=False),
        scratch_types=[
            pltpu.VMEM((SPW, NACT), jnp.int32),
            pltpu.VMEM((SPW, AD), jnp.float32),
        ],
    )
    def k(act_hbm, cnt_hbm, act_v, cnt_v):
        wid = lax.axis_index("s") * 2 + lax.axis_index("c")
        base = wid * SPW
        pltpu.sync_copy(act_hbm.at[pl.ds(base, SPW)], act_v)
        zeros16 = jnp.zeros((16,), jnp.float32)
        ones16 = jnp.ones((16,), jnp.float32)

        def zero_body(s, _):
            for v in range(AD // 16):
                cnt_v[s, pl.ds(v * 16, 16)] = zeros16
            return 0

        lax.fori_loop(0, SPW, zero_body, 0)

        def scat_body(s, _):
            svec = jnp.full((16,), s, jnp.int32)
            for v in range(NACT // 16):
                idx = act_v[s, pl.ds(v * 16, 16)]
                plsc.addupdate_scatter(cnt_v, [svec, idx], ones16)
            return 0

        lax.fori_loop(0, SPW, scat_body, 0)
        pltpu.sync_copy(cnt_v, cnt_hbm.at[pl.ds(base, SPW)])

    return k(active_as)


def _stats_body(c_ref, ft3, flb_ref, r1_ref, r2_ref, lb_ref, ftbuf, sems):
    q = pl.program_id(0)

    def ft_copy(qq, slot):
        return pltpu.make_async_copy(ft3.at[:, pl.ds(qq, 1), :],
                                     ftbuf.at[slot], sems.at[slot])

    @pl.when(q == 0)
    def _():
        for s in range(4):
            ft_copy(s, s).start()

    slot = lax.rem(q, 4)
    ft_copy(q, slot).wait()
    fb = ftbuf[slot].reshape(B, AD)
    cb = c_ref[...]
    t = cb * fb
    ones = jnp.ones((1, B), jnp.float32)
    dn = (((1,), (0,)), ((), ()))
    r1_ref[0] = lax.dot_general(ones, t, dn,
                                preferred_element_type=jnp.float32)
    r2_ref[0] = lax.dot_general(ones, t * fb, dn,
                                preferred_element_type=jnp.float32)

    @pl.when(q <= NF - 5)
    def _():
        ft_copy(q + 4, slot).start()

    @pl.when(q == NF - 1)
    def _():
        flb = flb_ref[...]
        lb_ref[0:1, :] = jnp.sum(flb, axis=0, keepdims=True)
        lb_ref[1:2, :] = jnp.sum(flb * flb, axis=0, keepdims=True)


def _stats_call(c2d, ft3, flb):
    return pl.pallas_call(
        _stats_body,
        grid=(NF,),
        in_specs=[
            pl.BlockSpec((B, AD), lambda q: (0, 0)),
            pl.BlockSpec(memory_space=pl.ANY),
            pl.BlockSpec((B, NLB), lambda q: (0, 0)),
        ],
        out_specs=[
            pl.BlockSpec((1, 1, AD), lambda q: (q, 0, 0)),
            pl.BlockSpec((1, 1, AD), lambda q: (q, 0, 0)),
            pl.BlockSpec((2, NLB), lambda q: (0, 0)),
        ],
        out_shape=[
            jax.ShapeDtypeStruct((NF, 1, AD), jnp.float32),
            jax.ShapeDtypeStruct((NF, 1, AD), jnp.float32),
            jax.ShapeDtypeStruct((2, NLB), jnp.float32),
        ],
        scratch_shapes=[
            pltpu.VMEM((4, B, 1, AD), jnp.float32),
            pltpu.SemaphoreType.DMA((4,)),
        ],
    )(c2d, ft3, flb)


def _mm_body(asm, bsm, ft3, m, flb, act, alb, blb, ahd, bhd, b1r, lnw, lnb,
             w4, b4r, w1r, out_ref, acc, wbuf, ftbuf, wlb, wact, sems,
             ftsems, semlb, semact):
    k = pl.program_id(0)
    dn = (((1,), (0,)), ((), ()))

    def wk_copy(kk, slot):
        return pltpu.make_async_copy(
            w1r.at[pl.ds(2, AD), pl.ds(kk, 1), :], wbuf.at[slot],
            sems.at[slot])

    def ft_copy(kk, slot):
        return pltpu.make_async_copy(ft3.at[:, pl.ds(kk, 1), :],
                                     ftbuf.at[slot], ftsems.at[slot])

    @pl.when(k == 0)
    def _():
        acc[...] = jnp.zeros((B, HID), jnp.float32)
        for s in range(4):
            wk_copy(s, s).start()
            ft_copy(s, s).start()
        pltpu.make_async_copy(w1r.at[pl.ds(0, 2), :, :], wlb, semlb).start()
        pltpu.make_async_copy(w1r.at[pl.ds(RW1 - 8, 8), :, :], wact,
                              semact).start()

    @pl.when(k <= NF - 1)
    def _():
        slot = lax.rem(k, 4)
        wk_copy(k, slot).wait()
        ft_copy(k, slot).wait()
        a = asm[k]
        b = bsm[k]
        x = m[...] * (ftbuf[slot].reshape(B, AD) * a + b)
        wv = wbuf[slot].reshape(AD, HID)
        acc[...] += lax.dot_general(x.astype(jnp.bfloat16),
                                    wv.astype(jnp.bfloat16), dn,
                                    preferred_element_type=jnp.float32)

        @pl.when(k <= NF - 5)
        def _():
            wk_copy(k + 4, slot).start()
            ft_copy(k + 4, slot).start()

    @pl.when(k == NF)
    def _():
        pltpu.make_async_copy(w1r.at[pl.ds(0, 2), :, :], wlb, semlb).wait()
        x0 = jnp.concatenate(
            [flb[:, NF:] * alb[...] + blb[...],
             flb[:, :NF] * ahd[...] + bhd[...]], axis=1)
        wv = wlb[...].reshape(NLB, HID)
        acc[...] += lax.dot_general(x0.astype(jnp.bfloat16),
                                    wv.astype(jnp.bfloat16), dn,
                                    preferred_element_type=jnp.float32)

    @pl.when(k == NK - 1)
    def _():
        pltpu.make_async_copy(w1r.at[pl.ds(RW1 - 8, 8), :, :], wact,
                              semact).wait()
        wv = wact[...].reshape(AD, HID)
        acc[...] += lax.dot_general(act[...].astype(jnp.bfloat16),
                                    wv.astype(jnp.bfloat16), dn,
                                    preferred_element_type=jnp.float32)
        h = acc[...] + b1r[...]
        h = jnp.where(h > 0, h, jnp.exp(jnp.minimum(h, 0.0)) - 1.0)
        mu = jnp.mean(h, axis=1, keepdims=True)
        hc = h - mu
        var = jnp.mean(hc * hc, axis=1, keepdims=True)
        hn = hc * lax.rsqrt(var + 1e-5) * lnw[...] + lnb[...]
        out_ref[...] = lax.dot_general(
            hn.astype(jnp.bfloat16), w4[...].astype(jnp.bfloat16), dn,
            preferred_element_type=jnp.float32) + b4r[...]


def _mm_call(alpha, beta, ft3, m, flb, act, alb, blb, ahd, bhd, b1r, lnw,
             lnb, W4, b4r, W1r):
    def full(shape):
        return pl.BlockSpec(shape, lambda k: tuple(0 for _ in shape))

    return pl.pallas_call(
        _mm_body,
        grid=(NK,),
        in_specs=[
            pl.BlockSpec(memory_space=pltpu.SMEM),
            pl.BlockSpec(memory_space=pltpu.SMEM),
            pl.BlockSpec(memory_space=pl.ANY),
            full((B, AD)),
            full((B, NLB)),
            full((B, AD)),
            full((1, NF)),
            full((1, NF)),
            full((1, NF)),
            full((1, NF)),
            full((1, HID)),
            full((1, HID)),
            full((1, HID)),
            full((HID, AD)),
            full((1, AD)),
            pl.BlockSpec(memory_space=pl.ANY),
        ],
        out_specs=pl.BlockSpec((B, AD), lambda k: (0, 0)),
        out_shape=jax.ShapeDtypeStruct((B, AD), jnp.float32),
        scratch_shapes=[
            pltpu.VMEM((B, HID), jnp.float32),
            pltpu.VMEM((4, AD, 1, HID), jnp.float32),
            pltpu.VMEM((4, B, 1, AD), jnp.float32),
            pltpu.VMEM((2, NF, HID), jnp.float32),
            pltpu.VMEM((8, NF, HID), jnp.float32),
            pltpu.SemaphoreType.DMA((4,)),
            pltpu.SemaphoreType.DMA((4,)),
            pltpu.SemaphoreType.DMA,
            pltpu.SemaphoreType.DMA,
        ],
    )(alpha, beta, ft3, m, flb, act, alb, blb, ahd, bhd, b1r, lnw, lnb,
      W4, b4r, W1r)


def kernel(feature_lb, feature_as, action, active_as, bn_as_w, bn_as_b,
           bn_lb_w, bn_lb_b, W1, b1, ln1_w, ln1_b, W4, b4):
    c2d = _sc_counts(active_as)

    ft3 = jnp.transpose(feature_as, (0, 2, 1))
    r1, r2, lbs = _stats_call(c2d, ft3, feature_lb)

    S1 = jnp.sum(r1.reshape(NF, AD), axis=1)
    S2 = jnp.sum(r2.reshape(NF, AD), axis=1)
    n_as = jnp.float32(B + B * NACT)
    mean_as = (S1 + lbs[0, :NF]) / n_as
    var_as = (S2 + lbs[1, :NF]) / n_as - mean_as * mean_as
    alpha_as = bn_as_w * lax.rsqrt(var_as + 1e-5)
    beta_as = bn_as_b - mean_as * alpha_as

    mean_lb = lbs[0, NF:] / B
    var_lb = lbs[1, NF:] / B - mean_lb * mean_lb
    alpha_lb = bn_lb_w * lax.rsqrt(var_lb + 1e-5)
    beta_lb = bn_lb_b - mean_lb * alpha_lb

    m = jnp.minimum(c2d, 1.0)
    W1r = W1.reshape(RW1, NF, HID)

    return _mm_call(alpha_as, beta_as, ft3, m, feature_lb, action,
                    alpha_lb[None, :], beta_lb[None, :], alpha_as[None, :],
                    beta_as[None, :], b1[None, :], ln1_w[None, :],
                    ln1_b[None, :], W4, b4[None, :], W1r)
